# baseline (speedup 1.0000x reference)
import functools

import jax
import jax.numpy as jnp
from jax import lax
from jax.experimental import pallas as pl
from jax.experimental.pallas import tpu as pltpu
from jax.experimental.pallas import tpu_sc as plsc

_L = 16
_R = 32


def _sc_body(x_hbm, tab_hbm, out_hbm, xv, tv, semx, semt):
    info = plsc.get_sparse_core_info()
    nc, ns = info.num_cores, info.num_subcores
    nw = nc * ns
    total_rows, d_model = x_hbm.shape
    seq = tab_hbm.shape[0]
    rows_per_w = total_rows // nw
    wid = lax.axis_index("s") * nc + lax.axis_index("c")
    base = wid * rows_per_w

    def chunk(i, carry):
        r0 = base + i * _R
        t0 = lax.rem(r0, seq)
        cx = pltpu.async_copy(x_hbm.at[pl.ds(r0, _R)], xv, semx)
        ct = pltpu.async_copy(tab_hbm.at[pl.ds(t0, _R)], tv, semt)
        cx.wait()
        ct.wait()

        def row(j, c2):
            for k in range(d_model // _L):
                sl = pl.ds(k * _L, _L)
                plsc.addupdate(xv.at[j, sl], tv[j, sl])
            return c2

        lax.fori_loop(0, _R, row, 0)
        pltpu.sync_copy(xv, out_hbm.at[pl.ds(r0, _R)])
        return carry

    lax.fori_loop(0, rows_per_w // _R, chunk, 0)


def kernel(x, pos_table):
    batch, seq_len, d_model = x.shape
    x2d = x.reshape(batch * seq_len, d_model)
    tab = pos_table[:seq_len]
    mesh = plsc.VectorSubcoreMesh(core_axis_name="c", subcore_axis_name="s")
    k = functools.partial(
        pl.kernel,
        mesh=mesh,
        out_type=jax.ShapeDtypeStruct((batch * seq_len, d_model), jnp.float32),
        scratch_types=[
            pltpu.VMEM((_R, d_model), jnp.float32),
            pltpu.VMEM((_R, d_model), jnp.float32),
            pltpu.SemaphoreType.DMA,
            pltpu.SemaphoreType.DMA,
        ],
    )(_sc_body)
    out2d = k(x2d, tab)
    return out2d.reshape(batch, seq_len, d_model)
